# baseline (speedup 1.0000x reference)
import functools

import jax
import jax.numpy as jnp
from jax import lax
from jax.experimental import pallas as pl
from jax.experimental.pallas import tpu as pltpu
from jax.experimental.pallas import tpu_sc as plsc

NUM_NODES = 100000
H_DIM = 128
CHUNK = 128
NCHUNK = (NUM_NODES + CHUNK - 1) // CHUNK
TAIL = NUM_NODES - (NCHUNK - 1) * CHUNK
NW = 32
SLOTS = 25
BIG = NCHUNK // NW + 1
NBIGW = NCHUNK - NW * (BIG - 1)
LAST_START = (NW - 1) * (BIG - 1) + NBIGW
LASTN = NUM_NODES - LAST_START * CHUNK
NBUF = 7
WD = 2


def _gather_body(idx_hbm, table_hbm, out_hbm, idx_v, rows_v, gsems, wsems):
    wid = lax.axis_index("s") * 2 + lax.axis_index("c")
    start = wid * (BIG - 1) + jnp.minimum(wid, NBIGW)
    n_full = jnp.where(wid < NBIGW, BIG, BIG - 1)
    n_full = jnp.where(wid == NW - 1, BIG - 2, n_full)
    last_w = wid == NW - 1

    @pl.when(jnp.logical_not(last_w))
    def _():
        pltpu.sync_copy(
            idx_hbm.at[pl.ds(start * CHUNK, SLOTS * CHUNK)], idx_v)

    @pl.when(last_w)
    def _():
        pltpu.sync_copy(idx_hbm.at[pl.ds(LAST_START * CHUNK, LASTN)],
                        idx_v.at[pl.ds(0, LASTN)])

    def gdesc(s, b):
        return pltpu.make_async_copy(
            table_hbm.at[idx_v.at[pl.ds(s * CHUNK, CHUNK)]],
            rows_v.at[b], gsems.at[b])

    def wdesc(s, b):
        return pltpu.make_async_copy(
            rows_v.at[b], out_hbm.at[pl.ds((start + s) * CHUNK, CHUNK)],
            wsems.at[b])

    def fire(s):
        @pl.when(s < n_full)
        def _():
            gdesc(s, lax.rem(s, NBUF)).start()

    for k in range(NBUF):
        fire(jnp.int32(k))

    def body(s, carry):
        @pl.when(s >= WD)
        def _():
            wdesc(s - WD, lax.rem(s - WD, NBUF)).wait()
            fire(s - WD + NBUF)

        b = lax.rem(s, NBUF)
        gdesc(s, b).wait()
        wdesc(s, b).start()
        return carry

    lax.fori_loop(0, n_full, body, 0)

    for k in range(WD):
        wdesc(n_full - WD + k, lax.rem(n_full - WD + k, NBUF)).wait()

    @pl.when(last_w)
    def _():
        g = pltpu.make_async_copy(
            table_hbm.at[idx_v.at[pl.ds((BIG - 2) * CHUNK, TAIL)]],
            rows_v.at[0].at[pl.ds(0, TAIL)], gsems.at[0])
        g.start()
        g.wait()
        pltpu.sync_copy(rows_v.at[0].at[pl.ds(0, TAIL)],
                        out_hbm.at[pl.ds((NCHUNK - 1) * CHUNK, TAIL)])


_mesh = plsc.VectorSubcoreMesh(core_axis_name="c", subcore_axis_name="s")

_gather = functools.partial(
    pl.kernel,
    mesh=_mesh,
    out_type=jax.ShapeDtypeStruct((NUM_NODES, H_DIM), jnp.float32),
    scratch_types=[
        pltpu.VMEM((SLOTS * CHUNK,), jnp.int32),
        pltpu.VMEM((NBUF, CHUNK, H_DIM), jnp.float32),
        pltpu.SemaphoreType.DMA((NBUF,)),
        pltpu.SemaphoreType.DMA((NBUF,)),
    ],
)(_gather_body)


@jax.jit
def kernel(g, h, r, norm, W):
    idx = h.reshape(-1).astype(jnp.int32)
    return _gather(idx, W)


# speedup vs baseline: 1.0010x; 1.0010x over previous
import functools

import jax
import jax.numpy as jnp
from jax import lax
from jax.experimental import pallas as pl
from jax.experimental.pallas import tpu as pltpu
from jax.experimental.pallas import tpu_sc as plsc

NUM_NODES = 100000
H_DIM = 128
CHUNK = 128
NCHUNK = (NUM_NODES + CHUNK - 1) // CHUNK
TAIL = NUM_NODES - (NCHUNK - 1) * CHUNK
NW = 32
SLOTS = 25
BIG = NCHUNK // NW + 1
NBIGW = NCHUNK - NW * (BIG - 1)
LAST_START = (NW - 1) * (BIG - 1) + NBIGW
LASTN = NUM_NODES - LAST_START * CHUNK
NBUF = 7
WD = 2


def _gather_body(idx_hbm, table_hbm, out_hbm, idx_v, rows_v, gsems, wsems):
    wid = lax.axis_index("s") * 2 + lax.axis_index("c")
    start = wid * (BIG - 1) + jnp.minimum(wid, NBIGW)
    n_full = jnp.where(wid < NBIGW, BIG, BIG - 1)
    n_full = jnp.where(wid == NW - 1, BIG - 2, n_full)
    last_w = wid == NW - 1

    HEAD = 8 * CHUNK
    pltpu.sync_copy(idx_hbm.at[pl.ds(start * CHUNK, HEAD)],
                    idx_v.at[pl.ds(0, HEAD)])

    def gdesc(s, b):
        return pltpu.make_async_copy(
            table_hbm.at[idx_v.at[pl.ds(s * CHUNK, CHUNK)]],
            rows_v.at[b], gsems.at[b])

    def wdesc(s, b):
        return pltpu.make_async_copy(
            rows_v.at[b], out_hbm.at[pl.ds((start + s) * CHUNK, CHUNK)],
            wsems.at[b])

    def fire(s):
        @pl.when(s < n_full)
        def _():
            gdesc(s, lax.rem(s, NBUF)).start()

    for k in range(NBUF):
        fire(jnp.int32(k))

    @pl.when(jnp.logical_not(last_w))
    def _():
        pltpu.sync_copy(
            idx_hbm.at[pl.ds(start * CHUNK + HEAD, SLOTS * CHUNK - HEAD)],
            idx_v.at[pl.ds(HEAD, SLOTS * CHUNK - HEAD)])

    @pl.when(last_w)
    def _():
        pltpu.sync_copy(
            idx_hbm.at[pl.ds(LAST_START * CHUNK + HEAD, LASTN - HEAD)],
            idx_v.at[pl.ds(HEAD, LASTN - HEAD)])

    def body(s, carry):
        @pl.when(s >= WD)
        def _():
            wdesc(s - WD, lax.rem(s - WD, NBUF)).wait()
            fire(s - WD + NBUF)

        b = lax.rem(s, NBUF)
        gdesc(s, b).wait()
        wdesc(s, b).start()
        return carry

    lax.fori_loop(0, n_full, body, 0)

    for k in range(WD):
        wdesc(n_full - WD + k, lax.rem(n_full - WD + k, NBUF)).wait()

    @pl.when(last_w)
    def _():
        g = pltpu.make_async_copy(
            table_hbm.at[idx_v.at[pl.ds((BIG - 2) * CHUNK, TAIL)]],
            rows_v.at[0].at[pl.ds(0, TAIL)], gsems.at[0])
        g.start()
        g.wait()
        pltpu.sync_copy(rows_v.at[0].at[pl.ds(0, TAIL)],
                        out_hbm.at[pl.ds((NCHUNK - 1) * CHUNK, TAIL)])


_mesh = plsc.VectorSubcoreMesh(core_axis_name="c", subcore_axis_name="s")

_gather = functools.partial(
    pl.kernel,
    mesh=_mesh,
    out_type=jax.ShapeDtypeStruct((NUM_NODES, H_DIM), jnp.float32),
    scratch_types=[
        pltpu.VMEM((SLOTS * CHUNK,), jnp.int32),
        pltpu.VMEM((NBUF, CHUNK, H_DIM), jnp.float32),
        pltpu.SemaphoreType.DMA((NBUF,)),
        pltpu.SemaphoreType.DMA((NBUF,)),
    ],
)(_gather_body)


@jax.jit
def kernel(g, h, r, norm, W):
    idx = h.reshape(-1).astype(jnp.int32)
    return _gather(idx, W)
